# Initial kernel scaffold; baseline (speedup 1.0000x reference)
import jax
import jax.numpy as jnp
from jax.experimental import pallas as pl


def kernel(users, items, item_year, item_manu, item_part, user_emb, item_emb, emb_manu, emb_part, W_y1, b_y1, W_y2, b_y2, W_proj, b_proj, W_m1, b_m1, W_m2, b_m2, W_he, b_he, W_hi, b_hi, W_g, b_g):
    raise NotImplementedError("write your pallas kernel here")


# speedup vs baseline: 1.3032x; 1.3032x over previous
import functools

import jax
import jax.numpy as jnp
from jax import lax
from jax.experimental import pallas as pl
from jax.experimental.pallas import tpu as pltpu
from jax.experimental.pallas import tpu_sc as plsc

B = 16384
DIM = 64
MD = 32
PD = 32

NC = 2
NS = 16
NW = NC * NS
BPW = B // NW
CH = 128
NCH = BPW // CH


def _sc_gather_body(u_idx, i_idx, m_idx, p_idx,
                    user_emb, item_emb, emb_manu, emb_part,
                    out_u, out_i, out_m, out_p,
                    vu_idx, vi_idx, vm_idx, vp_idx,
                    ru, ri, rm, rp,
                    s0, s1, s2, s3):
    wid = lax.axis_index("c") * NS + lax.axis_index("s")
    base = wid * BPW
    row0 = wid * NCH

    pltpu.sync_copy(u_idx.at[pl.ds(row0, NCH)], vu_idx)
    pltpu.sync_copy(i_idx.at[pl.ds(row0, NCH)], vi_idx)
    pltpu.sync_copy(m_idx.at[pl.ds(row0, NCH)], vm_idx)
    pltpu.sync_copy(p_idx.at[pl.ds(row0, NCH)], vp_idx)

    copies = []
    for j in range(NCH):
        copies.append(pltpu.async_copy(
            user_emb.at[vu_idx.at[j]], ru.at[pl.ds(j * CH, CH)], s0))
        copies.append(pltpu.async_copy(
            item_emb.at[vi_idx.at[j]], ri.at[pl.ds(j * CH, CH)], s1))
        copies.append(pltpu.async_copy(
            emb_manu.at[vm_idx.at[j]], rm.at[pl.ds(j * CH, CH)], s2))
        copies.append(pltpu.async_copy(
            emb_part.at[vp_idx.at[j]], rp.at[pl.ds(j * CH, CH)], s3))
    for c in copies:
        c.wait()

    pltpu.sync_copy(ru, out_u.at[pl.ds(base, BPW)])
    pltpu.sync_copy(ri, out_i.at[pl.ds(base, BPW)])
    pltpu.sync_copy(rm, out_m.at[pl.ds(base, BPW)])
    pltpu.sync_copy(rp, out_p.at[pl.ds(base, BPW)])


def _make_sc_gather():
    return functools.partial(
        pl.kernel,
        mesh=plsc.VectorSubcoreMesh(core_axis_name="c", subcore_axis_name="s"),
        compiler_params=pltpu.CompilerParams(use_tc_tiling_on_sc=False),
        out_type=[
            jax.ShapeDtypeStruct((B, DIM), jnp.float32),
            jax.ShapeDtypeStruct((B, DIM), jnp.float32),
            jax.ShapeDtypeStruct((B, MD), jnp.float32),
            jax.ShapeDtypeStruct((B, PD), jnp.float32),
        ],
        scratch_types=[
            pltpu.VMEM((NCH, CH), jnp.int32),
            pltpu.VMEM((NCH, CH), jnp.int32),
            pltpu.VMEM((NCH, CH), jnp.int32),
            pltpu.VMEM((NCH, CH), jnp.int32),
            pltpu.VMEM((BPW, DIM), jnp.float32),
            pltpu.VMEM((BPW, DIM), jnp.float32),
            pltpu.VMEM((BPW, MD), jnp.float32),
            pltpu.VMEM((BPW, PD), jnp.float32),
            pltpu.SemaphoreType.DMA,
            pltpu.SemaphoreType.DMA,
            pltpu.SemaphoreType.DMA,
            pltpu.SemaphoreType.DMA,
        ],
    )(_sc_gather_body)


def _mlp_body(year, u, ic, m, p,
              Wy1, by1, Wy2, by2, Wp, bp, Wm1, bm1, Wm2, bm2, Who, bho,
              out):
    f32 = jnp.float32
    relu = lambda a: jnp.maximum(a, 0.0)
    y1 = relu(year[...] * Wy1[...] + by1[...])
    y = relu(jnp.dot(y1, Wy2[...], preferred_element_type=f32) + by2[...])
    cin = jnp.concatenate([y, m[...], p[...]], axis=1)
    cont = relu(jnp.dot(cin, Wp[...], preferred_element_type=f32) + bp[...])
    x = jnp.concatenate([u[...], ic[...], cont], axis=1)
    h1 = relu(jnp.dot(x, Wm1[...], preferred_element_type=f32) + bm1[...])
    h = relu(jnp.dot(h1, Wm2[...], preferred_element_type=f32) + bm2[...])
    out[...] = jnp.dot(h, Who[...], preferred_element_type=f32) + bho[...]


def kernel(users, items, item_year, item_manu, item_part,
           user_emb, item_emb, emb_manu, emb_part,
           W_y1, b_y1, W_y2, b_y2, W_proj, b_proj,
           W_m1, b_m1, W_m2, b_m2, W_he, b_he, W_hi, b_hi, W_g, b_g):
    i32 = jnp.int32
    u_idx = users.astype(i32).reshape(B // CH, CH)
    i_idx = items.astype(i32).reshape(B // CH, CH)
    m_idx = item_manu.astype(i32).reshape(B // CH, CH)
    p_idx = item_part.astype(i32).reshape(B // CH, CH)

    u_g, ic_g, m_g, p_g = _make_sc_gather()(
        u_idx, i_idx, m_idx, p_idx, user_emb, item_emb, emb_manu, emb_part)

    Who = jnp.concatenate([W_he, W_hi], axis=1)
    bho = jnp.concatenate([b_he, b_hi]).reshape(1, 2)

    bs = 2048
    grid = (B // bs,)
    row_spec = lambda d: pl.BlockSpec((bs, d), lambda gi: (gi, 0))
    full = lambda a: pl.BlockSpec(a.shape, lambda gi: (0,) * a.ndim)

    out2 = pl.pallas_call(
        _mlp_body,
        grid=grid,
        in_specs=[
            row_spec(1), row_spec(DIM), row_spec(DIM), row_spec(MD), row_spec(PD),
            full(W_y1), full(b_y1.reshape(1, -1)),
            full(W_y2), full(b_y2.reshape(1, -1)),
            full(W_proj), full(b_proj.reshape(1, -1)),
            full(W_m1), full(b_m1.reshape(1, -1)),
            full(W_m2), full(b_m2.reshape(1, -1)),
            full(Who), full(bho),
        ],
        out_specs=pl.BlockSpec((bs, 2), lambda gi: (gi, 0)),
        out_shape=jax.ShapeDtypeStruct((B, 2), jnp.float32),
    )(item_year, u_g, ic_g, m_g, p_g,
      W_y1, b_y1.reshape(1, -1), W_y2, b_y2.reshape(1, -1),
      W_proj, b_proj.reshape(1, -1), W_m1, b_m1.reshape(1, -1),
      W_m2, b_m2.reshape(1, -1), Who, bho)

    return (out2[:, 0:1], out2[:, 1:2])
